# baseline (speedup 1.0000x reference)
import functools

import jax
import jax.numpy as jnp
from jax import lax
from jax.experimental import pallas as pl
from jax.experimental.pallas import tpu as pltpu
from jax.experimental.pallas import tpu_sc as plsc

DEPTH = 256
LANES = 16
NUM_CORES = 2
NUM_SUBCORES = 16
NUM_WORKERS = NUM_CORES * NUM_SUBCORES

_ZCOLS = 1024
_ZBLOCK_ROWS = 2048


def _tc_zeros(out_len: int):
    assert out_len % _ZCOLS == 0
    rows = out_len // _ZCOLS
    assert rows % _ZBLOCK_ROWS == 0
    grid = rows // _ZBLOCK_ROWS

    def zero_body(out_ref):
        out_ref[...] = jnp.zeros_like(out_ref)

    return pl.pallas_call(
        zero_body,
        out_shape=jax.ShapeDtypeStruct((rows, _ZCOLS), jnp.float32),
        grid=(grid,),
        out_specs=pl.BlockSpec((_ZBLOCK_ROWS, _ZCOLS), lambda i: (i, 0)),
    )()


def _build_scatter_kernel(n_idx: int):
    assert n_idx % NUM_WORKERS == 0
    idx_per_worker = n_idx // NUM_WORKERS
    k = 128
    assert idx_per_worker % k == 0
    n_desc = idx_per_worker // k
    vec_per_row = k // LANES

    mesh = plsc.VectorSubcoreMesh(
        core_axis_name="c",
        subcore_axis_name="s",
        num_cores=NUM_CORES,
        num_subcores=NUM_SUBCORES,
    )

    @functools.partial(
        pl.kernel,
        out_type=(),
        mesh=mesh,
        compiler_params=pltpu.CompilerParams(needs_layout_passes=False),
        scratch_types=[
            pltpu.VMEM((idx_per_worker,), jnp.int32),
            pltpu.VMEM((n_desc, k), jnp.int32),
            pltpu.VMEM((k,), jnp.float32),
            pltpu.SemaphoreType.DMA,
        ],
    )
    def scatter_kernel(in_hbm, out_ref, idx_all, pos2d, ones_v, sem):
        wid = lax.axis_index("s") * NUM_CORES + lax.axis_index("c")
        in_base = wid * idx_per_worker
        out_base = wid * idx_per_worker * DEPTH

        lane_off = lax.iota(jnp.int32, 16) * DEPTH
        ones = jnp.full((LANES,), 1.0, jnp.float32)

        pltpu.sync_copy(in_hbm.at[pl.ds(in_base, idx_per_worker)], idx_all)
        for u in range(vec_per_row):
            ones_v[pl.ds(u * LANES, LANES)] = ones

        @pl.loop(0, n_desc)
        def _(r):
            for u in range(vec_per_row):
                iv = idx_all[pl.ds(r * k + u * LANES, LANES)]
                pv = (out_base + u * LANES * DEPTH) + lane_off + iv
                pv = pv + r * (k * DEPTH)
                pos2d[r, pl.ds(u * LANES, LANES)] = pv

        @pl.loop(0, n_desc)
        def _(r):
            pltpu.async_copy(ones_v, out_ref.at[pos2d.at[r]], sem)

        @pl.loop(0, n_desc)
        def _(r):
            pltpu.make_async_copy(ones_v, out_ref.at[pos2d.at[0]], sem).wait()

    return scatter_kernel


@jax.jit
def _one_hot_flat(flat_idx):
    n_idx = flat_idx.shape[0]
    zeros = _tc_zeros(n_idx * DEPTH)
    out_ref = jax.new_ref(zeros.reshape(n_idx * DEPTH))
    _build_scatter_kernel(n_idx)(flat_idx, out_ref)
    return out_ref[...]


def kernel(inputs):
    B, W, H, D = inputs.shape
    n_idx = B * W * H * D
    flat = inputs.astype(jnp.int32).reshape(n_idx)
    out = _one_hot_flat(flat)
    return out.reshape(B, W, H, DEPTH * D)


# speedup vs baseline: 1.2245x; 1.2245x over previous
import functools

import jax
import jax.numpy as jnp
from jax import lax
from jax.experimental import pallas as pl
from jax.experimental.pallas import tpu as pltpu
from jax.experimental.pallas import tpu_sc as plsc

DEPTH = 256
LANES = 16
NUM_CORES = 2
NUM_SUBCORES = 16
NUM_WORKERS = NUM_CORES * NUM_SUBCORES


def _build_scatter_kernel(n_idx: int):
    assert n_idx % NUM_WORKERS == 0
    idx_per_worker = n_idx // NUM_WORKERS
    k = 128
    assert idx_per_worker % k == 0
    n_desc = idx_per_worker // k
    vec_per_row = k // LANES

    mesh = plsc.VectorSubcoreMesh(
        core_axis_name="c",
        subcore_axis_name="s",
        num_cores=NUM_CORES,
        num_subcores=NUM_SUBCORES,
    )

    @functools.partial(
        pl.kernel,
        out_type=(),
        mesh=mesh,
        compiler_params=pltpu.CompilerParams(needs_layout_passes=False),
        scratch_types=[
            pltpu.VMEM((idx_per_worker,), jnp.int32),
            pltpu.VMEM((n_desc, k), jnp.int32),
            pltpu.VMEM((k,), jnp.float32),
            pltpu.SemaphoreType.DMA,
        ],
    )
    def scatter_kernel(in_hbm, out_ref, idx_all, pos2d, ones_v, sem):
        wid = lax.axis_index("s") * NUM_CORES + lax.axis_index("c")
        in_base = wid * idx_per_worker
        out_base = wid * idx_per_worker * DEPTH

        lane_off = lax.iota(jnp.int32, 16) * DEPTH
        ones = jnp.full((LANES,), 1.0, jnp.float32)

        pltpu.sync_copy(in_hbm.at[pl.ds(in_base, idx_per_worker)], idx_all)
        for u in range(vec_per_row):
            ones_v[pl.ds(u * LANES, LANES)] = ones

        @pl.loop(0, n_desc)
        def _(r):
            for u in range(vec_per_row):
                iv = idx_all[pl.ds(r * k + u * LANES, LANES)]
                pv = (out_base + u * LANES * DEPTH) + lane_off + iv
                pv = pv + r * (k * DEPTH)
                pos2d[r, pl.ds(u * LANES, LANES)] = pv

        @pl.loop(0, n_desc)
        def _(r):
            pltpu.async_copy(ones_v, out_ref.at[pos2d.at[r]], sem)

        @pl.loop(0, n_desc)
        def _(r):
            pltpu.make_async_copy(ones_v, out_ref.at[pos2d.at[0]], sem).wait()

    return scatter_kernel


@jax.jit
def _one_hot_flat(flat_idx):
    n_idx = flat_idx.shape[0]
    out_ref = jax.new_ref(jnp.zeros((n_idx * DEPTH,), jnp.float32))
    _build_scatter_kernel(n_idx)(flat_idx, out_ref)
    return out_ref[...]


def kernel(inputs):
    B, W, H, D = inputs.shape
    n_idx = B * W * H * D
    flat = inputs.astype(jnp.int32).reshape(n_idx)
    out = _one_hot_flat(flat)
    return out.reshape(B, W, H, DEPTH * D)


# speedup vs baseline: 1.7304x; 1.4131x over previous
import functools

import jax
import jax.numpy as jnp
from jax import lax
from jax.experimental import pallas as pl
from jax.experimental.pallas import tpu as pltpu
from jax.experimental.pallas import tpu_sc as plsc

DEPTH = 256
LANES = 16
NUM_CORES = 2
NUM_SUBCORES = 16
NUM_WORKERS = NUM_CORES * NUM_SUBCORES


def _build_scatter_kernel(n_idx: int):
    assert n_idx % NUM_WORKERS == 0
    idx_per_worker = n_idx // NUM_WORKERS
    idx_per_chunk = 128
    vec_per_chunk = idx_per_chunk // LANES
    floats_per_chunk = idx_per_chunk * DEPTH
    assert idx_per_worker % idx_per_chunk == 0
    n_chunks = idx_per_worker // idx_per_chunk
    assert n_chunks % 2 == 0
    out_len = n_idx * DEPTH

    mesh = plsc.VectorSubcoreMesh(
        core_axis_name="c",
        subcore_axis_name="s",
        num_cores=NUM_CORES,
        num_subcores=NUM_SUBCORES,
    )

    @functools.partial(
        pl.kernel,
        out_type=jax.ShapeDtypeStruct((out_len,), jnp.float32),
        mesh=mesh,
        compiler_params=pltpu.CompilerParams(needs_layout_passes=False),
        scratch_types=[
            pltpu.VMEM((floats_per_chunk,), jnp.float32),
            pltpu.VMEM((idx_per_worker,), jnp.int32),
            pltpu.VMEM_SHARED((NUM_SUBCORES, 2, floats_per_chunk),
                              jnp.float32),
            pltpu.SemaphoreType.DMA,
            pltpu.SemaphoreType.DMA,
            pltpu.SemaphoreType.DMA,
        ],
    )
    def scatter_kernel(in_hbm, out_hbm, tilebuf, idx_all, slots,
                       sem1, sem2a, sem2b):
        cid = lax.axis_index("c")
        sid = lax.axis_index("s")
        wid = sid * NUM_CORES + cid
        in_base = wid * idx_per_worker
        out_base = wid * idx_per_worker * DEPTH

        sem2 = (sem2a, sem2b)
        lane_off = lax.iota(jnp.int32, 16) * DEPTH
        ones = jnp.full((LANES,), 1.0, jnp.float32)
        zeros_v = jnp.zeros((LANES,), jnp.float32)

        pltpu.sync_copy(in_hbm.at[pl.ds(in_base, idx_per_worker)], idx_all)

        @pl.loop(0, floats_per_chunk // (LANES * 8))
        def _(i):
            for u in range(8):
                tilebuf[pl.ds((i * 8 + u) * LANES, LANES)] = zeros_v

        def scatter(c, val):
            for jv in range(vec_per_chunk):
                iv = idx_all[pl.ds(c * idx_per_chunk + jv * LANES, LANES)]
                pv = lane_off + (jv * LANES * DEPTH) + iv
                plsc.store_scatter(tilebuf, [pv], val)

        def stage_and_ship(c, b):
            pltpu.async_copy(tilebuf, slots.at[sid, b], sem1)
            pltpu.make_async_copy(tilebuf, slots.at[sid, b], sem1).wait()
            pltpu.async_copy(
                slots.at[sid, b],
                out_hbm.at[pl.ds(out_base + c * floats_per_chunk,
                                 floats_per_chunk)],
                sem2[b],
            )

        def slot_free(b):
            pltpu.make_async_copy(
                slots.at[sid, b],
                out_hbm.at[pl.ds(out_base, floats_per_chunk)],
                sem2[b],
            ).wait()

        scatter(0, ones)
        stage_and_ship(0, 0)
        scatter(0, zeros_v)
        scatter(1, ones)
        stage_and_ship(1, 1)

        @pl.loop(1, n_chunks // 2)
        def _(g):
            for b in range(2):
                c = 2 * g + b
                slot_free(b)
                scatter(c - 1, zeros_v)
                scatter(c, ones)
                stage_and_ship(c, b)

        slot_free(0)
        slot_free(1)

    return scatter_kernel


def kernel(inputs):
    B, W, H, D = inputs.shape
    n_idx = B * W * H * D
    flat = inputs.astype(jnp.int32).reshape(n_idx)
    out = _build_scatter_kernel(n_idx)(flat)
    return out.reshape(B, W, H, DEPTH * D)


# speedup vs baseline: 1.9276x; 1.1140x over previous
import functools

import jax
import jax.numpy as jnp
from jax import lax
from jax.experimental import pallas as pl
from jax.experimental.pallas import tpu as pltpu
from jax.experimental.pallas import tpu_sc as plsc

DEPTH = 256
LANES = 16
NUM_CORES = 2
NUM_SUBCORES = 16
NUM_WORKERS = NUM_CORES * NUM_SUBCORES


def _build_scatter_kernel(n_idx: int):
    assert n_idx % NUM_WORKERS == 0
    idx_per_worker = n_idx // NUM_WORKERS
    nbuf = 2
    idx_per_chunk = 192
    vec_per_chunk = idx_per_chunk // LANES
    floats_per_chunk = idx_per_chunk * DEPTH
    assert idx_per_worker % idx_per_chunk == 0
    n_chunks = idx_per_worker // idx_per_chunk
    assert n_chunks % nbuf == 0
    out_len = n_idx * DEPTH

    mesh = plsc.VectorSubcoreMesh(
        core_axis_name="c",
        subcore_axis_name="s",
        num_cores=NUM_CORES,
        num_subcores=NUM_SUBCORES,
    )

    @functools.partial(
        pl.kernel,
        out_type=jax.ShapeDtypeStruct((out_len,), jnp.float32),
        mesh=mesh,
        compiler_params=pltpu.CompilerParams(needs_layout_passes=False),
        scratch_types=(
            [pltpu.VMEM((floats_per_chunk,), jnp.float32)] * nbuf
            + [pltpu.VMEM((idx_per_worker,), jnp.int32)]
            + [pltpu.SemaphoreType.DMA] * nbuf
        ),
    )
    def scatter_kernel(in_hbm, out_hbm, *scratch):
        bufs = scratch[:nbuf]
        idx_all = scratch[nbuf]
        sems = scratch[nbuf + 1:]
        wid = lax.axis_index("s") * NUM_CORES + lax.axis_index("c")
        in_base = wid * idx_per_worker
        out_base = wid * idx_per_worker * DEPTH
        lane_off = lax.iota(jnp.int32, 16) * DEPTH
        ones = jnp.full((LANES,), 1.0, jnp.float32)
        zeros_v = jnp.zeros((LANES,), jnp.float32)

        pltpu.sync_copy(in_hbm.at[pl.ds(in_base, idx_per_worker)], idx_all)

        @pl.loop(0, floats_per_chunk // (LANES * 8))
        def _(i):
            for u in range(8):
                off = (i * 8 + u) * LANES
                for buf in bufs:
                    buf[pl.ds(off, LANES)] = zeros_v

        def scatter(buf, c, val):
            for jv in range(vec_per_chunk):
                iv = idx_all[pl.ds(c * idx_per_chunk + jv * LANES, LANES)]
                pv = lane_off + (jv * LANES * DEPTH) + iv
                plsc.store_scatter(buf, [pv], val)

        def emit(c, b):
            scatter(bufs[b], c, ones)
            pltpu.async_copy(
                bufs[b],
                out_hbm.at[pl.ds(out_base + c * floats_per_chunk,
                                 floats_per_chunk)],
                sems[b],
            )

        def drain(b):
            pltpu.make_async_copy(
                bufs[b],
                out_hbm.at[pl.ds(out_base, floats_per_chunk)],
                sems[b],
            ).wait()

        for b in range(nbuf):
            emit(b, b)

        @pl.loop(1, n_chunks // nbuf)
        def _(g):
            for b in range(nbuf):
                drain(b)
                scatter(bufs[b], nbuf * (g - 1) + b, zeros_v)
                emit(nbuf * g + b, b)

        for b in range(nbuf):
            drain(b)

    return scatter_kernel


def kernel(inputs):
    B, W, H, D = inputs.shape
    n_idx = B * W * H * D
    flat = inputs.astype(jnp.int32).reshape(n_idx)
    out = _build_scatter_kernel(n_idx)(flat)
    return out.reshape(B, W, H, DEPTH * D)


# speedup vs baseline: 1.9505x; 1.0119x over previous
import functools

import jax
import jax.numpy as jnp
from jax import lax
from jax.experimental import pallas as pl
from jax.experimental.pallas import tpu as pltpu
from jax.experimental.pallas import tpu_sc as plsc

DEPTH = 256
LANES = 16
NUM_CORES = 2
NUM_SUBCORES = 16
NUM_WORKERS = NUM_CORES * NUM_SUBCORES


def _build_scatter_kernel(n_idx: int):
    assert n_idx % NUM_WORKERS == 0
    idx_per_worker = n_idx // NUM_WORKERS
    nbuf = 2
    idx_per_chunk = 192
    vec_per_chunk = idx_per_chunk // LANES
    floats_per_chunk = idx_per_chunk * DEPTH
    assert idx_per_worker % idx_per_chunk == 0
    n_chunks = idx_per_worker // idx_per_chunk
    assert n_chunks % nbuf == 0
    out_len = n_idx * DEPTH

    mesh = plsc.VectorSubcoreMesh(
        core_axis_name="c",
        subcore_axis_name="s",
        num_cores=NUM_CORES,
        num_subcores=NUM_SUBCORES,
    )

    @functools.partial(
        pl.kernel,
        out_type=jax.ShapeDtypeStruct((out_len,), jnp.float32),
        mesh=mesh,
        compiler_params=pltpu.CompilerParams(needs_layout_passes=False),
        scratch_types=(
            [pltpu.VMEM((floats_per_chunk,), jnp.float32)] * nbuf
            + [pltpu.VMEM((idx_per_worker,), jnp.int32)]
            + [pltpu.SemaphoreType.DMA] * nbuf
        ),
    )
    def scatter_kernel(in_hbm, out_hbm, *scratch):
        bufs = scratch[:nbuf]
        idx_all = scratch[nbuf]
        sems = scratch[nbuf + 1:]
        wid = lax.axis_index("s") * NUM_CORES + lax.axis_index("c")
        in_base = wid * idx_per_worker
        out_base = wid * idx_per_worker * DEPTH
        lane_off = lax.iota(jnp.int32, 16) * DEPTH
        ones = jnp.full((LANES,), 1.0, jnp.float32)
        zeros_v = jnp.zeros((LANES,), jnp.float32)

        in_copy = pltpu.make_async_copy(
            in_hbm.at[pl.ds(in_base, idx_per_worker)], idx_all, sems[0])
        in_copy.start()

        def zero_fill(buf):
            @pl.loop(0, floats_per_chunk // (LANES * 8))
            def _(i):
                for u in range(8):
                    buf[pl.ds((i * 8 + u) * LANES, LANES)] = zeros_v

        zero_fill(bufs[0])
        in_copy.wait()

        def scatter(buf, c, val):
            for jv in range(vec_per_chunk):
                iv = idx_all[pl.ds(c * idx_per_chunk + jv * LANES, LANES)]
                pv = lane_off + (jv * LANES * DEPTH) + iv
                plsc.store_scatter(buf, [pv], val)

        def emit(c, b):
            scatter(bufs[b], c, ones)
            pltpu.async_copy(
                bufs[b],
                out_hbm.at[pl.ds(out_base + c * floats_per_chunk,
                                 floats_per_chunk)],
                sems[b],
            )

        def drain(b):
            pltpu.make_async_copy(
                bufs[b],
                out_hbm.at[pl.ds(out_base, floats_per_chunk)],
                sems[b],
            ).wait()

        for b in range(nbuf):
            emit(b, b)
            if b + 1 < nbuf:
                zero_fill(bufs[b + 1])

        @pl.loop(1, n_chunks // nbuf)
        def _(g):
            for b in range(nbuf):
                drain(b)
                scatter(bufs[b], nbuf * (g - 1) + b, zeros_v)
                emit(nbuf * g + b, b)

        for b in range(nbuf):
            drain(b)

    return scatter_kernel


def kernel(inputs):
    B, W, H, D = inputs.shape
    n_idx = B * W * H * D
    flat = inputs.astype(jnp.int32).reshape(n_idx)
    out = _build_scatter_kernel(n_idx)(flat)
    return out.reshape(B, W, H, DEPTH * D)
